# baseline (speedup 1.0000x reference)
import functools

import jax
import jax.numpy as jnp
from jax import lax
from jax.experimental import pallas as pl
from jax.experimental.pallas import tpu as pltpu
from jax.experimental.pallas import tpu_sc as plsc

_K = 2048
_NROWS = 128
_NCOLS = 32768
_NWORKERS = 32
_ROWS_PER_WORKER = _NROWS // _NWORKERS
_NSLICES = _NCOLS // 16


@functools.partial(
    pl.kernel,
    out_type=jax.ShapeDtypeStruct((_NROWS, 16), jnp.float32),
    scratch_types=[
        pltpu.VMEM((_NCOLS,), jnp.float32),
        pltpu.VMEM((_NCOLS,), jnp.int32),
        pltpu.VMEM((4096,), jnp.int32),
        pltpu.VMEM((256,), jnp.int32),
        pltpu.VMEM((16,), jnp.float32),
    ],
    mesh=plsc.VectorSubcoreMesh(core_axis_name="c", subcore_axis_name="s"),
    compiler_params=pltpu.CompilerParams(needs_layout_passes=False),
)
def _sc_thresholds(x_hbm, thr_hbm, row_v, key_v, hist_v, hist2_v, thr_v):
    wid = lax.axis_index("s") * 2 + lax.axis_index("c")
    lane = lax.iota(jnp.int32, 16)
    lane_base = lane * 256
    lane_base2 = lane * 16
    ones16 = jnp.ones((16,), jnp.int32)
    zeros16 = jnp.zeros((16,), jnp.int32)

    def suffix(v):
        return lax.rev(jnp.cumsum(lax.rev(v, (0,))), (0,))

    def splat_count(mask):
        return jnp.max(plsc.all_reduce_population_count(mask))

    def do_row(r, _):
        row = wid * _ROWS_PER_WORKER + r
        pltpu.sync_copy(x_hbm.at[row], row_v)

        prefix = jnp.int32(0)
        krem = jnp.int32(_K)
        for p in range(4):
            shift = 24 - 8 * p

            def zero_hist(j, _):
                hist_v[pl.ds(j * 16, 16)] = zeros16
                return 0

            lax.fori_loop(0, 256, zero_hist, 0)
            for j in range(16):
                hist2_v[pl.ds(j * 16, 16)] = zeros16

            if p == 0:
                def body(i, _):
                    xv = row_v[pl.ds(i * 16, 16)]
                    b = lax.bitcast_convert_type(xv, jnp.int32)
                    m = lax.shift_right_arithmetic(b, 31)
                    key = b ^ (m | jnp.int32(-(2**31)))
                    key_v[pl.ds(i * 16, 16)] = key
                    digit = lax.shift_right_logical(key, 24)
                    plsc.addupdate_scatter(hist_v, [lane_base + digit], ones16)
                    plsc.addupdate_scatter(
                        hist2_v,
                        [lane_base2 + lax.shift_right_logical(digit, 4)],
                        ones16)
                    return 0
            else:
                pfx = prefix

                def body(i, _):
                    key = key_v[pl.ds(i * 16, 16)]
                    act = lax.shift_right_logical(key, shift + 8) == pfx
                    digit = lax.shift_right_logical(key, shift) & 255
                    plsc.addupdate_scatter(
                        hist_v, [lane_base + digit], ones16, mask=act)
                    plsc.addupdate_scatter(
                        hist2_v,
                        [lane_base2 + lax.shift_right_logical(digit, 4)],
                        ones16, mask=act)
                    return 0

            lax.fori_loop(0, _NSLICES, body, 0)

            gt = zeros16
            for l in range(16):
                gt = gt + hist2_v[pl.ds(l * 16, 16)]
            jstar = splat_count(suffix(gt) >= krem) - 1
            hj = zeros16
            for l in range(16):
                hj = hj + hist_v[pl.ds(l * 256 + jstar * 16, 16)]
            excl = jnp.sum(jnp.where(lane > jstar, gt, 0))
            sfx_w = suffix(hj) + excl
            lstar = splat_count(sfx_w >= krem) - 1
            s_d = excl + jnp.sum(jnp.where(lane >= lstar, hj, 0))
            h_d = jnp.sum(jnp.where(lane == lstar, hj, 0))
            krem = krem - (s_d - h_d)
            prefix = (prefix * 256) + jstar * 16 + lstar

        ukey = jnp.broadcast_to(prefix, (16,))
        thr_bits = jnp.where(ukey < 0, ukey ^ jnp.int32(-(2**31)), ~ukey)
        thr_v[...] = lax.bitcast_convert_type(thr_bits, jnp.float32)
        pltpu.sync_copy(thr_v, thr_hbm.at[row])
        return 0

    lax.fori_loop(0, _ROWS_PER_WORKER, do_row, 0)


def _mask_block(x_ref, t_ref, o_ref):
    x = x_ref[...]
    thr = t_ref[:, 0:1]
    o_ref[...] = jnp.where(x >= thr, x, jnp.float32(0.0))


def _tc_mask(inputs, thr_bcast):
    r = 64
    return pl.pallas_call(
        _mask_block,
        grid=(_NROWS // r,),
        in_specs=[
            pl.BlockSpec((r, _NCOLS), lambda i: (i, 0)),
            pl.BlockSpec((r, 128), lambda i: (i, 0)),
        ],
        out_specs=pl.BlockSpec((r, _NCOLS), lambda i: (i, 0)),
        out_shape=jax.ShapeDtypeStruct((_NROWS, _NCOLS), jnp.float32),
    )(inputs, thr_bcast)


def kernel(inputs, k):
    del k
    thr = _sc_thresholds(inputs)
    thr_bcast = jnp.broadcast_to(thr[:, 0:1], (_NROWS, 128))
    return _tc_mask(inputs, thr_bcast)


# speedup vs baseline: 1.0923x; 1.0923x over previous
import functools

import jax
import jax.numpy as jnp
from jax import lax
from jax.experimental import pallas as pl
from jax.experimental.pallas import tpu as pltpu
from jax.experimental.pallas import tpu_sc as plsc

_K = 2048
_NROWS = 128
_NCOLS = 32768
_NWORKERS = 32
_ROWS_PER_WORKER = _NROWS // _NWORKERS
_NSLICES = _NCOLS // 16
_UNROLL = 4


@functools.partial(
    pl.kernel,
    out_type=jax.ShapeDtypeStruct((_NROWS, 16), jnp.float32),
    scratch_types=[
        pltpu.VMEM((_NCOLS,), jnp.float32),
        pltpu.VMEM((_NCOLS,), jnp.int32),
        pltpu.VMEM((4096,), jnp.int32),
        pltpu.VMEM((256,), jnp.int32),
        pltpu.VMEM((16,), jnp.float32),
    ],
    mesh=plsc.VectorSubcoreMesh(core_axis_name="c", subcore_axis_name="s"),
    compiler_params=pltpu.CompilerParams(needs_layout_passes=False),
)
def _sc_thresholds(x_hbm, thr_hbm, row_v, key_v, hist_v, htot_v, thr_v):
    wid = lax.axis_index("s") * 2 + lax.axis_index("c")
    lane = lax.iota(jnp.int32, 16)
    lane_base = lane * 256
    ones16 = jnp.ones((16,), jnp.int32)
    zeros16 = jnp.zeros((16,), jnp.int32)

    def suffix(v):
        return lax.rev(jnp.cumsum(lax.rev(v, (0,))), (0,))

    def splat_count(mask):
        return jnp.max(plsc.all_reduce_population_count(mask))

    def do_row(r, _):
        row = wid * _ROWS_PER_WORKER + r
        pltpu.sync_copy(x_hbm.at[row], row_v)

        prefix = jnp.int32(0)
        krem = jnp.int32(_K)
        for p in range(4):
            shift = 24 - 8 * p

            def zero_hist(j, _):
                for u in range(_UNROLL):
                    hist_v[pl.ds((j * _UNROLL + u) * 16, 16)] = zeros16
                return 0

            lax.fori_loop(0, 256 // _UNROLL, zero_hist, 0)

            if p == 0:
                def body(i, _):
                    for u in range(_UNROLL):
                        s = (i * _UNROLL + u) * 16
                        xv = row_v[pl.ds(s, 16)]
                        b = lax.bitcast_convert_type(xv, jnp.int32)
                        m = lax.shift_right_arithmetic(b, 31)
                        key = b ^ (m | jnp.int32(-(2**31)))
                        key_v[pl.ds(s, 16)] = key
                        digit = lax.shift_right_logical(key, 24)
                        plsc.addupdate_scatter(
                            hist_v, [lane_base + digit], ones16)
                    return 0
            else:
                pfx = prefix

                def body(i, _):
                    for u in range(_UNROLL):
                        s = (i * _UNROLL + u) * 16
                        key = key_v[pl.ds(s, 16)]
                        act = lax.shift_right_logical(key, shift + 8) == pfx
                        digit = lax.shift_right_logical(key, shift) & 255
                        plsc.addupdate_scatter(
                            hist_v, [lane_base + digit], ones16, mask=act)
                    return 0

            lax.fori_loop(0, _NSLICES // _UNROLL, body, 0)

            gt = zeros16
            for j in range(16):
                acc = hist_v[pl.ds(j * 16, 16)]
                for l in range(1, 16):
                    acc = acc + hist_v[pl.ds(l * 256 + j * 16, 16)]
                htot_v[pl.ds(j * 16, 16)] = acc
                gt = gt + jnp.where(lane == j, jnp.sum(acc), 0)
            jstar = splat_count(suffix(gt) >= krem) - 1
            hj = htot_v[pl.ds(jstar * 16, 16)]
            excl = jnp.sum(jnp.where(lane > jstar, gt, 0))
            sfx_w = suffix(hj) + excl
            lstar = splat_count(sfx_w >= krem) - 1
            s_d = excl + jnp.sum(jnp.where(lane >= lstar, hj, 0))
            h_d = jnp.sum(jnp.where(lane == lstar, hj, 0))
            krem = krem - (s_d - h_d)
            prefix = (prefix * 256) + jstar * 16 + lstar

        ukey = jnp.broadcast_to(prefix, (16,))
        thr_bits = jnp.where(ukey < 0, ukey ^ jnp.int32(-(2**31)), ~ukey)
        thr_v[...] = lax.bitcast_convert_type(thr_bits, jnp.float32)
        pltpu.sync_copy(thr_v, thr_hbm.at[row])
        return 0

    lax.fori_loop(0, _ROWS_PER_WORKER, do_row, 0)


def _mask_block(x_ref, t_ref, o_ref):
    x = x_ref[...]
    thr = t_ref[:, 0:1]
    o_ref[...] = jnp.where(x >= thr, x, jnp.float32(0.0))


def _tc_mask(inputs, thr_bcast):
    r = 64
    return pl.pallas_call(
        _mask_block,
        grid=(_NROWS // r,),
        in_specs=[
            pl.BlockSpec((r, _NCOLS), lambda i: (i, 0)),
            pl.BlockSpec((r, 128), lambda i: (i, 0)),
        ],
        out_specs=pl.BlockSpec((r, _NCOLS), lambda i: (i, 0)),
        out_shape=jax.ShapeDtypeStruct((_NROWS, _NCOLS), jnp.float32),
    )(inputs, thr_bcast)


def kernel(inputs, k):
    del k
    thr = _sc_thresholds(inputs)
    thr_bcast = jnp.broadcast_to(thr[:, 0:1], (_NROWS, 128))
    return _tc_mask(inputs, thr_bcast)


# speedup vs baseline: 3.8662x; 3.5395x over previous
import functools

import jax
import jax.numpy as jnp
from jax import lax
from jax.experimental import pallas as pl
from jax.experimental.pallas import tpu as pltpu
from jax.experimental.pallas import tpu_sc as plsc

_K = 2048
_NROWS = 128
_NCOLS = 32768
_NWORKERS = 32
_ROWS_PER_WORKER = _NROWS // _NWORKERS
_NSLICES = _NCOLS // 16
_UNROLL = 8


@functools.partial(
    pl.kernel,
    out_type=jax.ShapeDtypeStruct((_NROWS, 16), jnp.float32),
    scratch_types=[
        pltpu.VMEM((_NCOLS,), jnp.float32),
        pltpu.VMEM((_NCOLS,), jnp.int32),
        pltpu.VMEM((4368,), jnp.int32),
        pltpu.VMEM((256,), jnp.int32),
        pltpu.VMEM((16,), jnp.float32),
    ],
    mesh=plsc.VectorSubcoreMesh(core_axis_name="c", subcore_axis_name="s"),
    compiler_params=pltpu.CompilerParams(needs_layout_passes=False),
)
def _sc_thresholds(x_hbm, thr_hbm, row_v, key_v, hist_v, htot_v, thr_v):
    wid = lax.axis_index("s") * 2 + lax.axis_index("c")
    lane = lax.iota(jnp.int32, 16)
    lane_base = lane * 273
    ones16 = jnp.ones((16,), jnp.int32)
    zeros16 = jnp.zeros((16,), jnp.int32)

    def suffix(v):
        return lax.rev(jnp.cumsum(lax.rev(v, (0,))), (0,))

    def splat_count(mask):
        return jnp.max(plsc.all_reduce_population_count(mask))

    def do_row(r, _):
        row = wid * _ROWS_PER_WORKER + r
        pltpu.sync_copy(x_hbm.at[row], row_v)

        prefix = jnp.int32(0)
        krem = jnp.int32(_K)
        for p in range(4):
            shift = 24 - 8 * p

            @plsc.parallel_loop(0, 273, 1, unroll=3)
            def _(j):
                hist_v[pl.ds(j * 16, 16)] = zeros16

            if p == 0:
                @plsc.parallel_loop(0, _NSLICES, 1, unroll=_UNROLL)
                def _(i):
                    s = i * 16
                    xv = row_v[pl.ds(s, 16)]
                    b = lax.bitcast_convert_type(xv, jnp.int32)
                    m = lax.shift_right_arithmetic(b, 31)
                    key = b ^ (m | jnp.int32(-(2**31)))
                    key_v[pl.ds(s, 16)] = key
                    digit = lax.shift_right_logical(key, 24)
                    plsc.addupdate_scatter(hist_v, [lane_base + digit], ones16)
            else:
                pfx = prefix

                @plsc.parallel_loop(0, _NSLICES, 1, unroll=_UNROLL)
                def _(i):
                    s = i * 16
                    key = key_v[pl.ds(s, 16)]
                    act = lax.shift_right_logical(key, shift + 8) == pfx
                    digit = lax.shift_right_logical(key, shift) & 255
                    plsc.addupdate_scatter(
                        hist_v, [lane_base + digit], ones16, mask=act)

            gt = zeros16
            for j in range(16):
                acc = hist_v[pl.ds(j * 16, 16)]
                for l in range(1, 16):
                    acc = acc + hist_v[pl.ds(l * 273 + j * 16, 16)]
                htot_v[pl.ds(j * 16, 16)] = acc
                gt = gt + jnp.where(lane == j, jnp.sum(acc), 0)
            jstar = splat_count(suffix(gt) >= krem) - 1
            hj = htot_v[pl.ds(jstar * 16, 16)]
            excl = jnp.sum(jnp.where(lane > jstar, gt, 0))
            sfx_w = suffix(hj) + excl
            lstar = splat_count(sfx_w >= krem) - 1
            s_d = excl + jnp.sum(jnp.where(lane >= lstar, hj, 0))
            h_d = jnp.sum(jnp.where(lane == lstar, hj, 0))
            krem = krem - (s_d - h_d)
            prefix = (prefix * 256) + jstar * 16 + lstar

        ukey = jnp.broadcast_to(prefix, (16,))
        thr_bits = jnp.where(ukey < 0, ukey ^ jnp.int32(-(2**31)), ~ukey)
        thr_v[...] = lax.bitcast_convert_type(thr_bits, jnp.float32)
        pltpu.sync_copy(thr_v, thr_hbm.at[row])
        return 0

    lax.fori_loop(0, _ROWS_PER_WORKER, do_row, 0)


def _mask_block(x_ref, t_ref, o_ref):
    x = x_ref[...]
    thr = t_ref[:, 0:1]
    o_ref[...] = jnp.where(x >= thr, x, jnp.float32(0.0))


def _tc_mask(inputs, thr_bcast):
    r = 64
    return pl.pallas_call(
        _mask_block,
        grid=(_NROWS // r,),
        in_specs=[
            pl.BlockSpec((r, _NCOLS), lambda i: (i, 0)),
            pl.BlockSpec((r, 128), lambda i: (i, 0)),
        ],
        out_specs=pl.BlockSpec((r, _NCOLS), lambda i: (i, 0)),
        out_shape=jax.ShapeDtypeStruct((_NROWS, _NCOLS), jnp.float32),
    )(inputs, thr_bcast)


def kernel(inputs, k):
    del k
    thr = _sc_thresholds(inputs)
    thr_bcast = jnp.broadcast_to(thr[:, 0:1], (_NROWS, 128))
    return _tc_mask(inputs, thr_bcast)
